# baseline (speedup 1.0000x reference)
import functools

import jax
import jax.numpy as jnp
from jax import lax
from jax.experimental import pallas as pl
from jax.experimental.pallas import tpu as pltpu
from jax.experimental.pallas import tpu_sc as plsc

NC = 2
NS = 16
NW = NC * NS
D = 16
L = 16
W = 65536


def _depad_body(x_ref, o_ref):
    o_ref[...] = x_ref[...].reshape(D, W // 128, 128)


@functools.lru_cache(maxsize=None)
def _make_depad(NV):
    n_blk = -(-NV // W)
    return pl.pallas_call(
        _depad_body,
        grid=(n_blk,),
        in_specs=[pl.BlockSpec((D, W), lambda i: (0, i))],
        out_specs=pl.BlockSpec((D, W // 128, 128), lambda i: (0, i, 0)),
        out_shape=jax.ShapeDtypeStruct((D, n_blk * (W // 128), 128),
                                       jnp.float32),
    )


def _gather_body(idx_hbm, lines_hbm, out_hbm,
                 idx_v, lane_v, offs, stages, blk, sems,
                 *, b_per_w, m_lines):
    wid = lax.axis_index("s") * NC + lax.axis_index("c")
    base = wid * b_per_w
    pltpu.sync_copy(idx_hbm.at[pl.ds(base, b_per_w)], idx_v)
    for c in range(b_per_w // L):
        sl = pl.ds(c * L, L)
        i = idx_v[sl]
        lane_v[sl] = lax.bitwise_and(i, 127)
        idx_v[sl] = lax.shift_right_logical(i, 7)

    n = b_per_w // L
    rows = lax.iota(jnp.int32, L)

    def fire(c, p):
        base16 = idx_v[pl.ds(c * L, L)]
        for f in range(D):
            offs[p][pl.ds(f * L, L)] = base16 + (f * m_lines)
        return pltpu.async_copy(lines_hbm.at[offs[p]], stages[p], sems[p])

    h = [None, None]
    h[0] = fire(0, 0)
    for c in range(n):
        p = c & 1
        if c + 1 < n:
            h[1 - p] = fire(c + 1, 1 - p)
        h[p].wait()
        sl = pl.ds(c * L, L)
        lanes16 = lane_v[sl]
        for f in range(D):
            blk[f, sl] = plsc.load_gather(stages[p],
                                          [rows + (f * L), lanes16])
    pltpu.sync_copy(blk, out_hbm.at[:, pl.ds(base, b_per_w)])


@functools.lru_cache(maxsize=None)
def _make_gather(B, m_lines):
    assert B % (8 * NW) == 0
    b_per_w = B // NW
    mesh = plsc.VectorSubcoreMesh(core_axis_name="c", subcore_axis_name="s",
                                  num_cores=NC, num_subcores=NS)
    f32 = jnp.float32
    return pl.kernel(
        functools.partial(_gather_body, b_per_w=b_per_w, m_lines=m_lines),
        out_type=jax.ShapeDtypeStruct((D, B), f32),
        mesh=mesh,
        scratch_types=[
            pltpu.VMEM((b_per_w,), jnp.int32),
            pltpu.VMEM((b_per_w,), jnp.int32),
            [pltpu.VMEM((D * L,), jnp.int32) for _ in range(2)],
            [pltpu.VMEM((D * L, 128), f32) for _ in range(2)],
            pltpu.VMEM((D, b_per_w), f32),
            [pltpu.SemaphoreType.DMA for _ in range(2)],
        ],
        compiler_params=pltpu.CompilerParams(needs_layout_passes=False),
    )


def _mlp_body(ue_ref, ie_ref, um_ref, im_ref, W1_ref, b1_ref, W2_ref, b2_ref,
              W3_ref, b3_ref, Wa_ref, ba_ref, out_ref):
    f32 = jnp.float32
    dn0 = (((0,), (0,)), ((), ()))

    x = jnp.concatenate([ue_ref[...], ie_ref[...]], axis=0)
    h = jnp.maximum(lax.dot_general(W1_ref[...], x, dn0,
                                    preferred_element_type=f32) + b1_ref[...],
                    0.0)
    h = jnp.maximum(lax.dot_general(W2_ref[...], h, dn0,
                                    preferred_element_type=f32) + b2_ref[...],
                    0.0)
    h = jnp.maximum(lax.dot_general(W3_ref[...], h, dn0,
                                    preferred_element_type=f32) + b3_ref[...],
                    0.0)
    mf = um_ref[...] * im_ref[...]
    v = jnp.concatenate([h, mf], axis=0)
    out_ref[...] = lax.dot_general(Wa_ref[...], v, dn0,
                                   preferred_element_type=f32) + ba_ref[...]


def kernel(user_indices, item_indices, emb_user_mlp, emb_item_mlp,
           emb_user_mf, emb_item_mf, W1, b1, W2, b2, W3, b3, Wa, ba):
    B = user_indices.shape[0]
    NV = emb_user_mlp.shape[0]
    uidx = user_indices.astype(jnp.int32)
    iidx = item_indices.astype(jnp.int32)

    depad = _make_depad(NV)
    m_lines = (-(-NV // W)) * (W // 128)
    gather = _make_gather(B, m_lines)

    acts = []
    for table, idx in ((emb_user_mlp, uidx), (emb_item_mlp, iidx),
                       (emb_user_mf, uidx), (emb_item_mf, iidx)):
        lines = depad(table.T).reshape(D * m_lines, 128)
        acts.append(gather(idx, lines))
    ue, ie, um, im = acts

    BLK = 4096
    grid = B // BLK
    act_spec = pl.BlockSpec((D, BLK), lambda i: (0, i))

    def w_spec(shape):
        return pl.BlockSpec(shape, lambda i: tuple(0 for _ in shape))

    out = pl.pallas_call(
        _mlp_body,
        grid=(grid,),
        in_specs=[
            act_spec, act_spec, act_spec, act_spec,
            w_spec((32, 32)), w_spec((32, 1)), w_spec((32, 16)),
            w_spec((16, 1)), w_spec((16, 8)), w_spec((8, 1)),
            w_spec((24, 1)), w_spec((1, 1)),
        ],
        out_specs=pl.BlockSpec((1, BLK), lambda i: (0, i)),
        out_shape=jax.ShapeDtypeStruct((1, B), jnp.float32),
    )(ue, ie, um, im,
      W1, b1.reshape(-1, 1), W2, b2.reshape(-1, 1), W3, b3.reshape(-1, 1),
      Wa, ba.reshape(-1, 1))
    return out.reshape(B, 1)


# speedup vs baseline: 1.3279x; 1.3279x over previous
import functools

import jax
import jax.numpy as jnp
from jax import lax
from jax.experimental import pallas as pl
from jax.experimental.pallas import tpu as pltpu
from jax.experimental.pallas import tpu_sc as plsc

NC = 2
NS = 16
NW = NC * NS
D = 16
L = 16
W = 32768


def _depad_body(a_ref, b_ref, o_ref):
    a = a_ref[...].reshape(D, W // 64, 64)
    b = b_ref[...].reshape(D, W // 64, 64)
    o_ref[...] = jnp.concatenate([a, b], axis=2)


@functools.lru_cache(maxsize=None)
def _make_depad(NV):
    n_blk = -(-NV // W)
    spec = pl.BlockSpec((D, W), lambda i: (0, i))
    return pl.pallas_call(
        _depad_body,
        grid=(n_blk,),
        in_specs=[spec, spec],
        out_specs=pl.BlockSpec((D, W // 64, 128), lambda i: (0, i, 0)),
        out_shape=jax.ShapeDtypeStruct((D, n_blk * (W // 64), 128),
                                       jnp.float32),
    )


def _gather_body(idx_hbm, lines_hbm, outa_hbm, outb_hbm,
                 idx_v, lane_v, offs, stages, blka, blkb, sems,
                 *, b_per_w, m_lines):
    wid = lax.axis_index("s") * NC + lax.axis_index("c")
    base = wid * b_per_w
    pltpu.sync_copy(idx_hbm.at[pl.ds(base, b_per_w)], idx_v)
    for c in range(b_per_w // L):
        sl = pl.ds(c * L, L)
        i = idx_v[sl]
        lane_v[sl] = lax.bitwise_and(i, 63)
        idx_v[sl] = lax.shift_right_logical(i, 6)

    n = b_per_w // L
    rows = lax.iota(jnp.int32, L)

    def fire(c, p):
        base16 = idx_v[pl.ds(c * L, L)]
        for f in range(D):
            offs[p][pl.ds(f * L, L)] = base16 + (f * m_lines)
        return pltpu.async_copy(lines_hbm.at[offs[p]], stages[p], sems[p])

    h = [None, None]
    h[0] = fire(0, 0)
    for c in range(n):
        p = c & 1
        if c + 1 < n:
            h[1 - p] = fire(c + 1, 1 - p)
        h[p].wait()
        sl = pl.ds(c * L, L)
        lanes16 = lane_v[sl]
        for f in range(D):
            blka[f, sl] = plsc.load_gather(stages[p],
                                           [rows + (f * L), lanes16])
            blkb[f, sl] = plsc.load_gather(stages[p],
                                           [rows + (f * L), lanes16 + 64])
    pltpu.sync_copy(blka, outa_hbm.at[:, pl.ds(base, b_per_w)])
    pltpu.sync_copy(blkb, outb_hbm.at[:, pl.ds(base, b_per_w)])


@functools.lru_cache(maxsize=None)
def _make_gather(B, m_lines):
    assert B % (8 * NW) == 0
    b_per_w = B // NW
    mesh = plsc.VectorSubcoreMesh(core_axis_name="c", subcore_axis_name="s",
                                  num_cores=NC, num_subcores=NS)
    f32 = jnp.float32
    out = jax.ShapeDtypeStruct((D, B), f32)
    return pl.kernel(
        functools.partial(_gather_body, b_per_w=b_per_w, m_lines=m_lines),
        out_type=(out, out),
        mesh=mesh,
        scratch_types=[
            pltpu.VMEM((b_per_w,), jnp.int32),
            pltpu.VMEM((b_per_w,), jnp.int32),
            [pltpu.VMEM((D * L,), jnp.int32) for _ in range(2)],
            [pltpu.VMEM((D * L, 128), f32) for _ in range(2)],
            pltpu.VMEM((D, b_per_w), f32),
            pltpu.VMEM((D, b_per_w), f32),
            [pltpu.SemaphoreType.DMA for _ in range(2)],
        ],
        compiler_params=pltpu.CompilerParams(needs_layout_passes=False),
    )


def _mlp_body(ue_ref, ie_ref, um_ref, im_ref, W1_ref, b1_ref, W2_ref, b2_ref,
              W3_ref, b3_ref, Wa_ref, ba_ref, out_ref):
    f32 = jnp.float32
    dn0 = (((0,), (0,)), ((), ()))

    x = jnp.concatenate([ue_ref[...], ie_ref[...]], axis=0)
    h = jnp.maximum(lax.dot_general(W1_ref[...], x, dn0,
                                    preferred_element_type=f32) + b1_ref[...],
                    0.0)
    h = jnp.maximum(lax.dot_general(W2_ref[...], h, dn0,
                                    preferred_element_type=f32) + b2_ref[...],
                    0.0)
    h = jnp.maximum(lax.dot_general(W3_ref[...], h, dn0,
                                    preferred_element_type=f32) + b3_ref[...],
                    0.0)
    mf = um_ref[...] * im_ref[...]
    v = jnp.concatenate([h, mf], axis=0)
    out_ref[...] = lax.dot_general(Wa_ref[...], v, dn0,
                                   preferred_element_type=f32) + ba_ref[...]


def kernel(user_indices, item_indices, emb_user_mlp, emb_item_mlp,
           emb_user_mf, emb_item_mf, W1, b1, W2, b2, W3, b3, Wa, ba):
    B = user_indices.shape[0]
    NV = emb_user_mlp.shape[0]
    uidx = user_indices.astype(jnp.int32)
    iidx = item_indices.astype(jnp.int32)

    depad = _make_depad(NV)
    m_lines = (-(-NV // W)) * (W // 64)
    gather = _make_gather(B, m_lines)

    lines_u = depad(emb_user_mlp.T, emb_user_mf.T).reshape(D * m_lines, 128)
    ue, um = gather(uidx, lines_u)
    lines_i = depad(emb_item_mlp.T, emb_item_mf.T).reshape(D * m_lines, 128)
    ie, im = gather(iidx, lines_i)

    BLK = 4096
    grid = B // BLK
    act_spec = pl.BlockSpec((D, BLK), lambda i: (0, i))

    def w_spec(shape):
        return pl.BlockSpec(shape, lambda i: tuple(0 for _ in shape))

    out = pl.pallas_call(
        _mlp_body,
        grid=(grid,),
        in_specs=[
            act_spec, act_spec, act_spec, act_spec,
            w_spec((32, 32)), w_spec((32, 1)), w_spec((32, 16)),
            w_spec((16, 1)), w_spec((16, 8)), w_spec((8, 1)),
            w_spec((24, 1)), w_spec((1, 1)),
        ],
        out_specs=pl.BlockSpec((1, BLK), lambda i: (0, i)),
        out_shape=jax.ShapeDtypeStruct((1, B), jnp.float32),
    )(ue, ie, um, im,
      W1, b1.reshape(-1, 1), W2, b2.reshape(-1, 1), W3, b3.reshape(-1, 1),
      Wa, ba.reshape(-1, 1))
    return out.reshape(B, 1)
